# Initial kernel scaffold; baseline (speedup 1.0000x reference)
import jax
import jax.numpy as jnp
from jax.experimental import pallas as pl


def kernel(x, edge_index, W1, b1, W2, b2, W3, b3):
    raise NotImplementedError("write your pallas kernel here")


# speedup vs baseline: 16.2644x; 16.2644x over previous
import functools

import jax
import jax.numpy as jnp
from jax import lax
from jax.experimental import pallas as pl
from jax.experimental.pallas import tpu as pltpu
from jax.experimental.pallas import tpu_sc as plsc

N = 10000
E = 320000
NP = 10240
PT = NP // 16
CH = 128
NW = 32
NCH = 79
EPT = NCH * CH
EP = EPT * NW

_mesh = plsc.VectorSubcoreMesh(core_axis_name="c", subcore_axis_name="s")
_sc_params = pltpu.CompilerParams(use_tc_tiling_on_sc=False)


@functools.partial(
    pl.kernel,
    mesh=_mesh,
    out_type=jax.ShapeDtypeStruct((2, NP), jnp.float32),
    compiler_params=_sc_params,
    scratch_types=[
        pltpu.VMEM((CH,), jnp.int32),
        pltpu.VMEM((CH,), jnp.float32),
        pltpu.VMEM_SHARED((NP,), jnp.float32),
    ],
)
def _deg(dst_hbm, ones_hbm, z_hbm, out_hbm, idx_d, ones_v, acc):
    c = lax.axis_index("c")
    s = lax.axis_index("s")
    wid = c * 16 + s
    pltpu.sync_copy(z_hbm, acc.at[pl.ds(s * PT, PT)])
    pltpu.sync_copy(ones_hbm, ones_v)
    plsc.subcore_barrier()
    base = wid * EPT

    def body(j, carry):
        off = pl.multiple_of(base + j * CH, CH)
        pltpu.sync_copy(dst_hbm.at[pl.ds(off, CH)], idx_d)
        pltpu.sync_copy(ones_v, acc.at[idx_d], add=True)
        return carry

    lax.fori_loop(0, NCH, body, 0)
    plsc.subcore_barrier()
    pltpu.sync_copy(acc.at[pl.ds(s * PT, PT)], out_hbm.at[c, pl.ds(s * PT, PT)])


def _make_agg(D):
    @functools.partial(
        pl.kernel,
        mesh=_mesh,
        out_type=jax.ShapeDtypeStruct((2, NP, D), jnp.float32),
        compiler_params=_sc_params,
        scratch_types=[
            pltpu.VMEM((CH,), jnp.int32),
            pltpu.VMEM((CH,), jnp.int32),
            pltpu.VMEM((CH, D), jnp.float32),
            pltpu.VMEM_SHARED((NP, D), jnp.float32),
            pltpu.SemaphoreType.DMA,
        ],
    )
    def agg(src_hbm, dst_hbm, g_hbm, z_hbm, out_hbm, idx_s, idx_d, rows, acc, sem):
        c = lax.axis_index("c")
        s = lax.axis_index("s")
        wid = c * 16 + s
        pltpu.sync_copy(z_hbm, acc.at[pl.ds(s * PT, PT)])
        plsc.subcore_barrier()
        base = wid * EPT

        def body(j, carry):
            off = pl.multiple_of(base + j * CH, CH)
            pltpu.sync_copy(src_hbm.at[pl.ds(off, CH)], idx_s)
            pltpu.sync_copy(dst_hbm.at[pl.ds(off, CH)], idx_d)
            pltpu.async_copy(g_hbm.at[idx_s], rows, sem).wait()
            pltpu.sync_copy(rows, acc.at[idx_d], add=True)
            return carry

        lax.fori_loop(0, NCH, body, 0)
        plsc.subcore_barrier()
        pltpu.sync_copy(
            acc.at[pl.ds(s * PT, PT)], out_hbm.at[c, pl.ds(s * PT, PT)]
        )

    return agg


_agg64 = _make_agg(64)
_agg32 = _make_agg(32)


def _tc1(x, W1, degT):
    def body(x_ref, w_ref, dg_ref, g1_ref, dinv_ref):
        dsum = dg_ref[:, 0:1] + dg_ref[:, 1:2] + 1.0
        dinv = lax.rsqrt(dsum[0:N, :])
        u = jnp.dot(x_ref[...], w_ref[...], preferred_element_type=jnp.float32)
        g1_ref[...] = u * dinv
        dinv_ref[...] = dinv

    return pl.pallas_call(
        body,
        out_shape=(
            jax.ShapeDtypeStruct((N, 64), jnp.float32),
            jax.ShapeDtypeStruct((N, 1), jnp.float32),
        ),
    )(x, W1, degT)


def _tc2(p0, p1, g1, dinv, W2, b1):
    def body(p0_ref, p1_ref, g1_ref, dinv_ref, w_ref, b_ref, g2_ref):
        agg = p0_ref[0:N, :] + p1_ref[0:N, :] + g1_ref[...]
        h1 = jnp.maximum(dinv_ref[...] * agg + b_ref[...], 0.0)
        u = jnp.dot(h1, w_ref[...], preferred_element_type=jnp.float32)
        g2_ref[...] = u * dinv_ref[...]

    return pl.pallas_call(
        body,
        out_shape=jax.ShapeDtypeStruct((N, 32), jnp.float32),
    )(p0, p1, g1, dinv, W2, b1)


def _tc3(q0, q1, g2, dinv, w3r, b2, b3):
    def body(q0_ref, q1_ref, g2_ref, dinv_ref, w_ref, b2_ref, b3_ref, o_ref):
        agg = q0_ref[0:N, :] + q1_ref[0:N, :] + g2_ref[...]
        h2 = jnp.maximum(dinv_ref[...] * agg + b2_ref[...], 0.0)
        o_ref[...] = (
            jnp.sum(h2 * w_ref[...], axis=1, keepdims=True) + b3_ref[...]
        )

    return pl.pallas_call(
        body,
        out_shape=jax.ShapeDtypeStruct((N, 1), jnp.float32),
    )(q0, q1, g2, dinv, w3r, b2, b3)


def kernel(x, edge_index, W1, b1, W2, b2, W3, b3):
    src = edge_index[0].astype(jnp.int32)
    dst = edge_index[1].astype(jnp.int32)
    pad = EP - E
    src_p = jnp.concatenate([src, jnp.zeros((pad,), jnp.int32)])
    dst_p = jnp.concatenate([dst, jnp.full((pad,), N, jnp.int32)])
    ones = jnp.ones((CH,), jnp.float32)
    z1 = jnp.zeros((PT,), jnp.float32)
    z64 = jnp.zeros((PT, 64), jnp.float32)
    z32 = jnp.zeros((PT, 32), jnp.float32)

    degp = _deg(dst_p, ones, z1)
    g1, dinv = _tc1(x, W1, degp.T)
    p = _agg64(src_p, dst_p, g1, z64)
    g2 = _tc2(p[0], p[1], g1, dinv, W2, b1.reshape(1, 64))
    q = _agg32(src_p, dst_p, g2, z32)
    out = _tc3(q[0], q[1], g2, dinv, W3.reshape(1, 32), b2.reshape(1, 32),
               b3.reshape(1, 1))
    return out


# speedup vs baseline: 19.7535x; 1.2145x over previous
import functools

import jax
import jax.numpy as jnp
from jax import lax
from jax.experimental import pallas as pl
from jax.experimental.pallas import tpu as pltpu
from jax.experimental.pallas import tpu_sc as plsc

N = 10000
E = 320000
NP = 10240
PT = NP // 16
CH = 128
NW = 32
NCH = 80
K = 4
NS = NCH // K
EPT = NCH * CH
EP = EPT * NW

_mesh = plsc.VectorSubcoreMesh(core_axis_name="c", subcore_axis_name="s")
_sc_params = pltpu.CompilerParams(use_tc_tiling_on_sc=False)


@functools.partial(
    pl.kernel,
    mesh=_mesh,
    out_type=jax.ShapeDtypeStruct((2, NP), jnp.float32),
    compiler_params=_sc_params,
    scratch_types=[
        pltpu.VMEM((NCH, CH), jnp.int32),
        pltpu.VMEM((CH,), jnp.float32),
        pltpu.VMEM_SHARED((NP,), jnp.float32),
        pltpu.SemaphoreType.DMA,
        pltpu.SemaphoreType.DMA,
    ],
)
def _deg(dst_hbm, ones_hbm, z_hbm, out_hbm, dbuf, ones_v, acc, s0, s1):
    c = lax.axis_index("c")
    s = lax.axis_index("s")
    wid = c * 16 + s
    pltpu.sync_copy(z_hbm, acc.at[pl.ds(s * PT, PT)])
    pltpu.sync_copy(ones_hbm, ones_v)
    pltpu.sync_copy(dst_hbm.at[pl.ds(wid * NCH, NCH)], dbuf)
    plsc.subcore_barrier()
    sems = [s0, s1]

    def fire_set(st, sem):
        for k in range(K * 2):
            pltpu.async_copy(ones_v, acc.at[dbuf.at[st * K * 2 + k]], sem,
                             add=True)

    def drain_set(sem):
        for _ in range(K * 2):
            pltpu.make_async_copy(ones_v, acc.at[dbuf.at[0]], sem).wait()

    nsets = NCH // (K * 2)
    fire_set(0, sems[0])
    fire_set(1, sems[1])

    def body(p, carry):
        drain_set(sems[0])
        fire_set(2 * p + 2, sems[0])
        drain_set(sems[1])
        fire_set(2 * p + 3, sems[1])
        return carry

    lax.fori_loop(0, nsets // 2 - 1, body, 0)
    drain_set(sems[0])
    drain_set(sems[1])
    plsc.subcore_barrier()
    pltpu.sync_copy(acc.at[pl.ds(s * PT, PT)], out_hbm.at[c, pl.ds(s * PT, PT)])


def _make_agg(D):
    rows_t = [pltpu.VMEM((CH, D), jnp.float32) for _ in range(2 * K)]

    @functools.partial(
        pl.kernel,
        mesh=_mesh,
        out_type=jax.ShapeDtypeStruct((2, NP, D), jnp.float32),
        compiler_params=_sc_params,
        scratch_types=[
            pltpu.VMEM((NCH, CH), jnp.int32),
            pltpu.VMEM((NCH, CH), jnp.int32),
            pltpu.VMEM_SHARED((NP, D), jnp.float32),
            pltpu.SemaphoreType.DMA,
            pltpu.SemaphoreType.DMA,
            pltpu.SemaphoreType.DMA,
            pltpu.SemaphoreType.DMA,
        ]
        + rows_t,
    )
    def agg(src_hbm, dst_hbm, g_hbm, z_hbm, out_hbm, sbuf, dbuf, acc,
            sg0, sg1, ss0, ss1, *rows):
        c = lax.axis_index("c")
        s = lax.axis_index("s")
        wid = c * 16 + s
        pltpu.sync_copy(z_hbm, acc.at[pl.ds(s * PT, PT)])
        pltpu.sync_copy(src_hbm.at[pl.ds(wid * NCH, NCH)], sbuf)
        pltpu.sync_copy(dst_hbm.at[pl.ds(wid * NCH, NCH)], dbuf)
        plsc.subcore_barrier()
        sg = [sg0, sg1]
        ss = [ss0, ss1]
        banks = [rows[0:K], rows[K:2 * K]]

        def fire_gathers(st, bank, sem):
            for k in range(K):
                pltpu.async_copy(g_hbm.at[sbuf.at[st * K + k]], bank[k], sem)

        def wait_gathers(bank, sem):
            for k in range(K):
                pltpu.make_async_copy(g_hbm.at[sbuf.at[0]], bank[k], sem).wait()

        def fire_scatters(st, bank, sem):
            for k in range(K):
                pltpu.async_copy(bank[k], acc.at[dbuf.at[st * K + k]], sem,
                                 add=True)

        def drain_scatters(bank, sem):
            for k in range(K):
                pltpu.make_async_copy(bank[k], acc.at[dbuf.at[0]], sem).wait()

        def half(t, parity, first, last):
            X, Y = parity, 1 - parity
            if not last:
                if not first:
                    drain_scatters(banks[Y], ss[Y])
                fire_gathers(t + 1, banks[Y], sg[Y])
            wait_gathers(banks[X], sg[X])
            fire_scatters(t, banks[X], ss[X])

        fire_gathers(0, banks[0], sg[0])
        half(0, 0, True, False)

        def body(p, carry):
            t = 2 * p + 1
            half(t, 1, False, False)
            half(t + 1, 0, False, False)
            return carry

        lax.fori_loop(0, (NS - 2) // 2, body, 0)
        half(NS - 1, 1, False, True)
        drain_scatters(banks[0], ss[0])
        drain_scatters(banks[1], ss[1])
        plsc.subcore_barrier()
        pltpu.sync_copy(
            acc.at[pl.ds(s * PT, PT)], out_hbm.at[c, pl.ds(s * PT, PT)]
        )

    return agg


_agg64 = _make_agg(64)
_agg32 = _make_agg(32)


def _tc1(x, W1, degT):
    def body(x_ref, w_ref, dg_ref, g1_ref, dinv_ref):
        dsum = dg_ref[:, 0:1] + dg_ref[:, 1:2] + 1.0
        dinv = lax.rsqrt(dsum[0:N, :])
        u = jnp.dot(x_ref[...], w_ref[...], preferred_element_type=jnp.float32)
        g1_ref[...] = u * dinv
        dinv_ref[...] = dinv

    return pl.pallas_call(
        body,
        out_shape=(
            jax.ShapeDtypeStruct((N, 64), jnp.float32),
            jax.ShapeDtypeStruct((N, 1), jnp.float32),
        ),
    )(x, W1, degT)


def _tc2(p0, p1, g1, dinv, W2, b1):
    def body(p0_ref, p1_ref, g1_ref, dinv_ref, w_ref, b_ref, g2_ref):
        agg = p0_ref[0:N, :] + p1_ref[0:N, :] + g1_ref[...]
        h1 = jnp.maximum(dinv_ref[...] * agg + b_ref[...], 0.0)
        u = jnp.dot(h1, w_ref[...], preferred_element_type=jnp.float32)
        g2_ref[...] = u * dinv_ref[...]

    return pl.pallas_call(
        body,
        out_shape=jax.ShapeDtypeStruct((N, 32), jnp.float32),
    )(p0, p1, g1, dinv, W2, b1)


def _tc3(q0, q1, g2, dinv, w3r, b2, b3):
    def body(q0_ref, q1_ref, g2_ref, dinv_ref, w_ref, b2_ref, b3_ref, o_ref):
        agg = q0_ref[0:N, :] + q1_ref[0:N, :] + g2_ref[...]
        h2 = jnp.maximum(dinv_ref[...] * agg + b2_ref[...], 0.0)
        o_ref[...] = (
            jnp.sum(h2 * w_ref[...], axis=1, keepdims=True) + b3_ref[...]
        )

    return pl.pallas_call(
        body,
        out_shape=jax.ShapeDtypeStruct((N, 1), jnp.float32),
    )(q0, q1, g2, dinv, w3r, b2, b3)


def kernel(x, edge_index, W1, b1, W2, b2, W3, b3):
    src = edge_index[0].astype(jnp.int32)
    dst = edge_index[1].astype(jnp.int32)
    pad = EP - E
    src_p = jnp.concatenate([src, jnp.zeros((pad,), jnp.int32)])
    dst_p = jnp.concatenate([dst, jnp.full((pad,), N, jnp.int32)])
    src2 = src_p.reshape(EP // CH, CH)
    dst2 = dst_p.reshape(EP // CH, CH)
    ones = jnp.ones((CH,), jnp.float32)
    z1 = jnp.zeros((PT,), jnp.float32)
    z64 = jnp.zeros((PT, 64), jnp.float32)
    z32 = jnp.zeros((PT, 32), jnp.float32)

    degp = _deg(dst2, ones, z1)
    g1, dinv = _tc1(x, W1, degp.T)
    p = _agg64(src2, dst2, g1, z64)
    g2 = _tc2(p[0], p[1], g1, dinv, W2, b1.reshape(1, 64))
    q = _agg32(src2, dst2, g2, z32)
    out = _tc3(q[0], q[1], g2, dinv, W3.reshape(1, 32), b2.reshape(1, 32),
               b3.reshape(1, 1))
    return out
